# baseline (speedup 1.0000x reference)
import functools

import jax
import jax.numpy as jnp
from jax import lax
from jax.experimental import pallas as pl
from jax.experimental.pallas import tpu as pltpu
from jax.experimental.pallas import tpu_sc as plsc

NUM_ROWS = 1_000_000
EMBED_K = 16
BATCH = 16384
NUM_WORKERS = 32
BPW = BATCH // NUM_WORKERS
LANES = 16

NBLK = 7813
LAST_BLK = NBLK - 1
LINE_ROWS = NBLK * 16
SUP_BLKS = 12
SUP_COLS = SUP_BLKS * 128
NSUP = (NBLK - 1) // SUP_BLKS
NITER = -(-NSUP // NUM_WORKERS)

BURST = 256
BURSTS = BPW // BURST
BCHUNKS = BURST // LANES

_mesh = plsc.VectorSubcoreMesh(core_axis_name="c", subcore_axis_name="s")


@functools.partial(
    pl.kernel,
    out_type=(jax.ShapeDtypeStruct((LINE_ROWS, 128), jnp.float32),
              jax.ShapeDtypeStruct((LINE_ROWS, 128), jnp.float32)),
    mesh=_mesh,
    compiler_params=pltpu.CompilerParams(needs_layout_passes=False),
    scratch_types=[
        pltpu.VMEM((16, SUP_COLS + 1), jnp.float32),
        pltpu.VMEM((16, SUP_COLS + 1), jnp.float32),
        pltpu.VMEM((16 * SUP_BLKS, 128), jnp.float32),
        pltpu.VMEM((16 * SUP_BLKS, 128), jnp.float32),
        pltpu.SemaphoreType.DMA,
        pltpu.SemaphoreType.DMA,
    ],
)
def _detile(wt_hbm, ht_hbm, wlin_hbm, hlin_hbm,
            b0, b1, t0, t1, sem_r, sem_w):
    wid = lax.axis_index("s") * 2 + lax.axis_index("c")
    lane = lax.iota(jnp.int32, LANES)
    bufs = (b0, b1)
    tbufs = (t0, t1)

    zero16 = jnp.zeros((LANES,), jnp.int32)

    def transpose_block_small(B, T):
        def row_body(t16, _):
            for m in range(8):
                T[t16, pl.ds(m * LANES, LANES)] = (
                    plsc.load_gather(B, [lane, zero16 + (8 * t16 + m)]))
            return 0

        lax.fori_loop(0, 16, row_body, 0)

    def transpose_block(B, T, blk, bcol):
        base = blk * 16

        def pair_body(p, _):
            r0 = base + 2 * p
            c0 = bcol + 16 * p
            for sub in range(2):
                for m in range(8):
                    T[r0 + sub, pl.ds(m * LANES, LANES)] = (
                        plsc.load_gather(B, [lane, zero16 + (c0 + 8 * sub + m)]))
            return 0

        lax.fori_loop(0, 8, pair_body, 0)

    def run_table(src_hbm, dst_hbm):
        def start_read(s, buf):
            @pl.when(s < NSUP)
            def _():
                pltpu.make_async_copy(
                    src_hbm.at[:, pl.ds(s * SUP_COLS, SUP_COLS)],
                    buf.at[:, pl.ds(0, SUP_COLS)], sem_r).start()

        def wait_read():
            pltpu.make_async_copy(
                src_hbm.at[:, pl.ds(0, SUP_COLS)],
                b0.at[:, pl.ds(0, SUP_COLS)], sem_r).wait()

        def wait_write():
            pltpu.make_async_copy(
                src_hbm.at[:, pl.ds(0, SUP_COLS)], t0, sem_w).wait()

        start_read(wid, bufs[0])

        def step(t, parity):
            s = t * NUM_WORKERS + wid
            start_read(s + NUM_WORKERS, bufs[1 - parity])
            B = bufs[parity]
            T = tbufs[parity]

            @pl.when(s < NSUP)
            def _():
                wait_read()

                @pl.when(t >= 2)
                def _():
                    wait_write()

                def blk_body(blk, _):
                    transpose_block(B, T, blk, blk * 128)
                    return 0

                lax.fori_loop(0, SUP_BLKS, blk_body, 0)
                pltpu.make_async_copy(
                    T, dst_hbm.at[pl.ds(s * 16 * SUP_BLKS, 16 * SUP_BLKS)],
                    sem_w).start()

        def body(u, _):
            for parity in (0, 1):
                step(2 * u + parity, parity)
            return 0

        lax.fori_loop(0, (NITER + 2) // 2, body, 0)
        wait_write()
        wait_write()

        @pl.when(wid == 0)
        def _():
            for k in range(16):
                pltpu.sync_copy(
                    src_hbm.at[k, pl.ds(LAST_BLK * 128, 64)],
                    b0.at[k, pl.ds(0, 64)])
            transpose_block_small(b0, t0)
            pltpu.sync_copy(t0.at[pl.ds(0, 16)],
                            dst_hbm.at[pl.ds(LAST_BLK * 16, 16)])

    run_table(wt_hbm, wlin_hbm)
    run_table(ht_hbm, hlin_hbm)


@functools.partial(
    pl.kernel,
    out_type=jax.ShapeDtypeStruct((BATCH,), jnp.float32),
    mesh=_mesh,
    compiler_params=pltpu.CompilerParams(needs_layout_passes=False),
    scratch_types=[
        pltpu.VMEM((BPW,), jnp.int32),
        pltpu.VMEM((BPW,), jnp.int32),
        pltpu.VMEM((BPW,), jnp.int32),
        pltpu.VMEM((BPW,), jnp.int32),
        pltpu.VMEM((BURST, 128), jnp.float32),
        pltpu.VMEM((BURST, 128), jnp.float32),
        pltpu.VMEM((BPW,), jnp.float32),
        pltpu.SemaphoreType.DMA,
        pltpu.SemaphoreType.DMA,
    ],
)
def _mf_predict(uidx_hbm, vidx_hbm, wlin_hbm, hlin_hbm, out_hbm,
                uidx_v, vidx_v, usamp_v, vsamp_v, u2d, v2d, o_v,
                sem_u, sem_v):
    wid = lax.axis_index("s") * 2 + lax.axis_index("c")
    base = wid * BPW

    pltpu.sync_copy(uidx_hbm.at[pl.ds(base, BPW)], uidx_v)
    pltpu.sync_copy(vidx_hbm.at[pl.ds(base, BPW)], vidx_v)

    def samp_body(c, _):
        off = pl.ds(c * LANES, LANES)
        usamp_v[off] = uidx_v[off] >> 3
        vsamp_v[off] = vidx_v[off] >> 3
        return 0

    lax.fori_loop(0, BPW // LANES, samp_body, 0)

    lane = lax.iota(jnp.int32, LANES)

    for b in range(BURSTS):
        boff = pl.ds(b * BURST, BURST)
        cu = pltpu.async_copy(wlin_hbm.at[usamp_v.at[boff]], u2d, sem_u)
        cv = pltpu.async_copy(hlin_hbm.at[vsamp_v.at[boff]], v2d, sem_v)
        cu.wait()
        cv.wait()

        def chunk_body(g, _):
            goff = pl.ds(b * BURST + g * LANES, LANES)
            rowv = g * LANES + lane
            ucol = (uidx_v[goff] & 7) << 4
            vcol = (vidx_v[goff] & 7) << 4
            acc = jnp.zeros((LANES,), jnp.float32)
            for k in range(EMBED_K):
                uw = plsc.load_gather(u2d, [rowv, ucol + k])
                vw = plsc.load_gather(v2d, [rowv, vcol + k])
                acc = acc + uw * vw
            o_v[goff] = 1.0 / (1.0 + jnp.exp(-acc))
            return 0

        lax.fori_loop(0, BCHUNKS, chunk_body, 0)

    pltpu.sync_copy(o_v, out_hbm.at[pl.ds(base, BPW)])


def kernel(x, W, H):
    uidx = x[:, 0].astype(jnp.int32)
    vidx = x[:, 1].astype(jnp.int32)
    wlin, hlin = _detile(W.T, H.T)
    return _mf_predict(uidx, vidx, wlin, hlin)


# speedup vs baseline: 1.0305x; 1.0305x over previous
import functools

import jax
import jax.numpy as jnp
from jax import lax
from jax.experimental import pallas as pl
from jax.experimental.pallas import tpu as pltpu
from jax.experimental.pallas import tpu_sc as plsc

NUM_ROWS = 1_000_000
EMBED_K = 16
BATCH = 16384
NUM_WORKERS = 32
BPW = BATCH // NUM_WORKERS
LANES = 16

NBLK = 7813
LAST_BLK = NBLK - 1
LINE_ROWS = NBLK * 16
SUP_BLKS = 12
SUP_COLS = SUP_BLKS * 128
NSUP = (NBLK - 1) // SUP_BLKS
NITER = -(-NSUP // NUM_WORKERS)

BURST = 256
BURSTS = BPW // BURST
BCHUNKS = BURST // LANES

_mesh = plsc.VectorSubcoreMesh(core_axis_name="c", subcore_axis_name="s")


@functools.partial(
    pl.kernel,
    out_type=(jax.ShapeDtypeStruct((LINE_ROWS, 128), jnp.float32),
              jax.ShapeDtypeStruct((LINE_ROWS, 128), jnp.float32)),
    mesh=_mesh,
    compiler_params=pltpu.CompilerParams(needs_layout_passes=False),
    scratch_types=[
        pltpu.VMEM((16, SUP_COLS + 1), jnp.float32),
        pltpu.VMEM((16, SUP_COLS + 1), jnp.float32),
        pltpu.VMEM((16 * SUP_BLKS, 128), jnp.float32),
        pltpu.VMEM((16 * SUP_BLKS, 128), jnp.float32),
        pltpu.SemaphoreType.DMA,
        pltpu.SemaphoreType.DMA,
    ],
)
def _detile(wt_hbm, ht_hbm, wlin_hbm, hlin_hbm,
            b0, b1, t0, t1, sem_r, sem_w):
    wid = lax.axis_index("s") * 2 + lax.axis_index("c")
    lane = lax.iota(jnp.int32, LANES)
    bufs = (b0, b1)
    tbufs = (t0, t1)

    zero16 = jnp.zeros((LANES,), jnp.int32)

    def transpose_block_small(B, T):
        def row_body(t16, _):
            for m in range(8):
                T[t16, pl.ds(m * LANES, LANES)] = (
                    plsc.load_gather(B, [lane, zero16 + (8 * t16 + m)]))
            return 0

        lax.fori_loop(0, 16, row_body, 0)

    def transpose_block(B, T, blk, bcol):
        base = blk * 16

        def pair_body(p, _):
            r0 = base + 2 * p
            c0 = bcol + 16 * p
            for sub in range(2):
                for m in range(8):
                    T[r0 + sub, pl.ds(m * LANES, LANES)] = (
                        plsc.load_gather(B, [lane, zero16 + (c0 + 8 * sub + m)]))
            return 0

        lax.fori_loop(0, 8, pair_body, 0)

    def run_table(src_hbm, dst_hbm):
        def start_read(s, buf):
            @pl.when(s < NSUP)
            def _():
                pltpu.make_async_copy(
                    src_hbm.at[:, pl.ds(s * SUP_COLS, SUP_COLS)],
                    buf.at[:, pl.ds(0, SUP_COLS)], sem_r).start()

        def wait_read():
            pltpu.make_async_copy(
                src_hbm.at[:, pl.ds(0, SUP_COLS)],
                b0.at[:, pl.ds(0, SUP_COLS)], sem_r).wait()

        def wait_write():
            pltpu.make_async_copy(
                src_hbm.at[:, pl.ds(0, SUP_COLS)], t0, sem_w).wait()

        start_read(wid, bufs[0])

        def step(t, parity):
            s = t * NUM_WORKERS + wid
            start_read(s + NUM_WORKERS, bufs[1 - parity])
            B = bufs[parity]
            T = tbufs[parity]

            @pl.when(s < NSUP)
            def _():
                wait_read()

                @pl.when(t >= 2)
                def _():
                    wait_write()

                def blk_body(blk, _):
                    transpose_block(B, T, blk, blk * 128)
                    return 0

                lax.fori_loop(0, SUP_BLKS, blk_body, 0)
                pltpu.make_async_copy(
                    T, dst_hbm.at[pl.ds(s * 16 * SUP_BLKS, 16 * SUP_BLKS)],
                    sem_w).start()

        def body(u, _):
            for parity in (0, 1):
                step(2 * u + parity, parity)
            return 0

        lax.fori_loop(0, (NITER + 2) // 2, body, 0)
        wait_write()
        wait_write()

        @pl.when(wid == 0)
        def _():
            for k in range(16):
                pltpu.sync_copy(
                    src_hbm.at[k, pl.ds(LAST_BLK * 128, 64)],
                    b0.at[k, pl.ds(0, 64)])
            transpose_block_small(b0, t0)
            pltpu.sync_copy(t0.at[pl.ds(0, 16)],
                            dst_hbm.at[pl.ds(LAST_BLK * 16, 16)])

    run_table(wt_hbm, wlin_hbm)
    run_table(ht_hbm, hlin_hbm)


@functools.partial(
    pl.kernel,
    out_type=jax.ShapeDtypeStruct((BATCH,), jnp.float32),
    mesh=_mesh,
    compiler_params=pltpu.CompilerParams(needs_layout_passes=False),
    scratch_types=[
        pltpu.VMEM((BPW,), jnp.int32),
        pltpu.VMEM((BPW,), jnp.int32),
        pltpu.VMEM((BPW,), jnp.int32),
        pltpu.VMEM((BPW,), jnp.int32),
        pltpu.VMEM((BURST, 128), jnp.float32),
        pltpu.VMEM((BURST, 128), jnp.float32),
        pltpu.VMEM((BPW,), jnp.float32),
        pltpu.SemaphoreType.DMA,
        pltpu.SemaphoreType.DMA,
    ],
)
def _mf_predict(uidx_hbm, vidx_hbm, wlin_hbm, hlin_hbm, out_hbm,
                uidx_v, vidx_v, usamp_v, vsamp_v, u2d, v2d, o_v,
                sem_u, sem_v):
    wid = lax.axis_index("s") * 2 + lax.axis_index("c")
    base = wid * BPW

    pltpu.sync_copy(uidx_hbm.at[pl.ds(base, BPW)], uidx_v)
    pltpu.sync_copy(vidx_hbm.at[pl.ds(base, BPW)], vidx_v)

    def samp_body(c, _):
        off = pl.ds(c * LANES, LANES)
        usamp_v[off] = uidx_v[off] >> 3
        vsamp_v[off] = vidx_v[off] >> 3
        return 0

    lax.fori_loop(0, BPW // LANES, samp_body, 0)

    lane = lax.iota(jnp.int32, LANES)

    for b in range(BURSTS):
        boff = pl.ds(b * BURST, BURST)
        cu = pltpu.async_copy(wlin_hbm.at[usamp_v.at[boff]], u2d, sem_u)
        cv = pltpu.async_copy(hlin_hbm.at[vsamp_v.at[boff]], v2d, sem_v)
        cu.wait()
        cv.wait()

        def chunk_body(g, _):
            goff = pl.ds(b * BURST + g * LANES, LANES)
            rowv = g * LANES + lane
            ucol = (uidx_v[goff] & 7) << 4
            vcol = (vidx_v[goff] & 7) << 4
            acc = jnp.zeros((LANES,), jnp.float32)
            for k in range(EMBED_K):
                uw = plsc.load_gather(u2d, [rowv, ucol + k])
                vw = plsc.load_gather(v2d, [rowv, vcol + k])
                acc = acc + uw * vw
            o_v[goff] = 1.0 / (1.0 + jnp.exp(-acc))
            return 0

        lax.fori_loop(0, BCHUNKS, chunk_body, 0)

    pltpu.sync_copy(o_v, out_hbm.at[pl.ds(base, BPW)])


def kernel(x, W, H):
    uidx = x[:, 0].astype(jnp.int32)
    vidx = x[:, 1].astype(jnp.int32)
    return _mf_predict(uidx, vidx,
                       W.reshape(NUM_ROWS // 8, 128),
                       H.reshape(NUM_ROWS // 8, 128))
